# baseline (speedup 1.0000x reference)
import jax
import jax.numpy as jnp
from jax import lax
from jax.experimental import pallas as pl
from jax.experimental.pallas import tpu as pltpu
from jax.experimental.pallas import tpu_sc as plsc

N = 4096
D = 256
H = 256
KNN = 16
BQ = 256
TW = 2 * H + 128
NBLK = N // BQ

_SC_CORES = 2
_SC_SUBCORES = 16
_NW = _SC_CORES * _SC_SUBCORES
_BPW = (N * KNN) // _NW
_CH = 64


_IMAX = 0x7FFFFFFF
_NG = 32
_NL = 128
_SDEPTH = 4


def _table_body(feat_b, cp16_b, Wq, Wk, Wv, table_b, q_b):
    f = feat_b[...]
    q_b[...] = jnp.dot(f, Wq[...], preferred_element_type=jnp.float32)
    table_b[:, 0:H] = jnp.dot(f, Wk[...], preferred_element_type=jnp.float32)
    table_b[:, H:2 * H] = jnp.dot(f, Wv[...], preferred_element_type=jnp.float32)
    table_b[:, 2 * H:2 * H + 16] = cp16_b[...]


def _topk_body(cp_b, cpT, idx_b):
    c = cp_b[...]
    ct = cpT[...]
    g = jnp.dot(c, ct, preferred_element_type=jnp.float32)
    sqb = jnp.sum(c * c, axis=1, keepdims=True)
    sqr = jnp.sum(ct * ct, axis=0, keepdims=True)
    d = sqb + sqr - 2.0 * g
    bits = lax.bitcast_convert_type(jnp.maximum(d, 0.0), jnp.int32)
    d3 = bits.reshape(BQ, _NG, _NL)
    giota = lax.broadcasted_iota(jnp.int32, (BQ, _NG, _NL), 1)
    cur = d3
    svals, sgidx = [], []
    for _ in range(_SDEPTH):
        m = jnp.min(cur, axis=1)
        ag = jnp.min(jnp.where(cur == m[:, None, :], giota, _NG), axis=1)
        svals.append(m)
        sgidx.append(ag)
        cur = jnp.where(giota == ag[:, None, :], _IMAX, cur)
    m5 = jnp.min(cur, axis=1)
    s0, s1, s2, s3 = svals
    g0, g1, g2, g3 = sgidx
    liota = lax.broadcasted_iota(jnp.int32, (BQ, _NL), 1)
    cnt = jnp.zeros((BQ, _NL), jnp.int32)
    cols = []
    mn = None
    for _ in range(KNN):
        mn = jnp.min(s0, axis=1, keepdims=True)
        gi = jnp.min(jnp.where(s0 == mn, g0 * _NL + liota, N),
                     axis=1, keepdims=True)
        win = liota == (gi & (_NL - 1))
        cols.append(gi)
        s0 = jnp.where(win, s1, s0)
        g0 = jnp.where(win, g1, g0)
        s1 = jnp.where(win, s2, s1)
        g1 = jnp.where(win, g2, g1)
        s2 = jnp.where(win, s3, s2)
        g2 = jnp.where(win, g3, g2)
        s3 = jnp.where(win, _IMAX, s3)
        cnt = cnt + win.astype(jnp.int32)
    idx_b[...] = jnp.concatenate(cols, axis=1)
    bad = jnp.any((cnt >= _SDEPTH) & (m5 <= mn))

    @pl.when(bad)
    def _fallback():
        dd = bits
        iota = lax.broadcasted_iota(jnp.int32, (BQ, N), 1)
        fcols = []
        for _ in range(KNN):
            fmn = jnp.min(dd, axis=1, keepdims=True)
            am = jnp.min(jnp.where(dd == fmn, iota, N), axis=1, keepdims=True)
            fcols.append(am)
            dd = jnp.where(iota == am, _IMAX, dd)
        idx_b[...] = jnp.concatenate(fcols, axis=1)


def _stage_c_body(G_b, q_b, cp_b, feat_b, Wp1p, bp1, Wp2, bp2,
                  Wg1, bg1, Wg2, bg2, out_b):
    Gv = G_b[...]
    kj = Gv[:, 0:H]
    vj = Gv[:, H:2 * H]
    cj = Gv[:, 2 * H:2 * H + 8]
    cq = cp_b[...]
    rel = (cq[:, None, :] - cj.reshape(BQ, KNN, 8)).reshape(BQ * KNN, 8)
    t = jnp.maximum(
        jnp.dot(rel, Wp1p[...], preferred_element_type=jnp.float32) + bp1[...],
        0.0)
    delta = jnp.dot(t, Wp2[...], preferred_element_type=jnp.float32) + bp2[...]
    qv = q_b[...]
    gg = (qv[:, None, :] - kj.reshape(BQ, KNN, H)
          + delta.reshape(BQ, KNN, H)).reshape(BQ * KNN, H)
    h = jnp.maximum(
        jnp.dot(gg, Wg1[...], preferred_element_type=jnp.float32) + bg1[...],
        0.0)
    logits = (jnp.dot(h, Wg2[...], preferred_element_type=jnp.float32)
              + bg2[...]).reshape(BQ, KNN, H)
    m = jnp.max(logits, axis=1, keepdims=True)
    e = jnp.exp(logits - m)
    s = jnp.sum(e, axis=1, keepdims=True)
    attn = e / s
    wv = vj.reshape(BQ, KNN, H) + delta.reshape(BQ, KNN, H)
    out_b[...] = feat_b[...] + jnp.sum(attn * wv, axis=1)


_NSPLIT = 4
_HB = NBLK // _NSPLIT
_HROWS = (N // _NSPLIT) * KNN
_BPW = _HROWS // _NW


def _sc_gather_body(table_hbm, idx_hbm, out_hbm, idx_v, buf, sem):
    wid = lax.axis_index("s") * _SC_CORES + lax.axis_index("c")
    base = wid * _BPW
    pltpu.sync_copy(idx_hbm.at[pl.ds(base, _BPW)], idx_v)

    def chunk(c, carry):
        off = c * _CH
        pltpu.async_copy(table_hbm.at[idx_v.at[pl.ds(off, _CH)]],
                         buf, sem).wait()
        pltpu.sync_copy(buf, out_hbm.at[pl.ds(base + off, _CH)])
        return carry

    lax.fori_loop(0, _BPW // _CH, chunk, 0)


def _sc_gather(table, idx_flat):
    mesh = plsc.VectorSubcoreMesh(core_axis_name="c", subcore_axis_name="s")
    f = pl.kernel(
        _sc_gather_body,
        mesh=mesh,
        out_type=jax.ShapeDtypeStruct((_HROWS, TW), jnp.float32),
        scratch_types=[
            pltpu.VMEM((_BPW,), jnp.int32),
            pltpu.VMEM((_CH, TW), jnp.float32),
            pltpu.SemaphoreType.DMA,
        ],
    )
    return f(table, idx_flat)


def _topk_half(cp8, cpT, half):
    return pl.pallas_call(
        _topk_body,
        grid=(_HB,),
        in_specs=[
            pl.BlockSpec((BQ, 8), lambda i: (i + half * _HB, 0)),
            pl.BlockSpec((8, N), lambda i: (0, 0)),
        ],
        out_specs=pl.BlockSpec((BQ, KNN), lambda i: (i, 0)),
        out_shape=jax.ShapeDtypeStruct((N // _NSPLIT, KNN), jnp.int32),
    )(cp8, cpT)


def _attn_half(gathered, q, cp8, feat, Wp1p, bp1, bp2, Wp2, Wg1, bg1,
               Wg2, bg2, half):
    return pl.pallas_call(
        _stage_c_body,
        grid=(_HB,),
        in_specs=[
            pl.BlockSpec((BQ * KNN, TW), lambda i: (i, 0)),
            pl.BlockSpec((BQ, H), lambda i: (i + half * _HB, 0)),
            pl.BlockSpec((BQ, 8), lambda i: (i + half * _HB, 0)),
            pl.BlockSpec((BQ, D), lambda i: (i + half * _HB, 0)),
            pl.BlockSpec((8, H), lambda i: (0, 0)),
            pl.BlockSpec((1, H), lambda i: (0, 0)),
            pl.BlockSpec((H, H), lambda i: (0, 0)),
            pl.BlockSpec((1, H), lambda i: (0, 0)),
            pl.BlockSpec((H, H), lambda i: (0, 0)),
            pl.BlockSpec((1, H), lambda i: (0, 0)),
            pl.BlockSpec((H, H), lambda i: (0, 0)),
            pl.BlockSpec((1, H), lambda i: (0, 0)),
        ],
        out_specs=pl.BlockSpec((BQ, H), lambda i: (i, 0)),
        out_shape=jax.ShapeDtypeStruct((N // _NSPLIT, H), jnp.float32),
    )(gathered, q, cp8, feat,
      Wp1p, bp1, Wp2, bp2, Wg1, bg1, Wg2, bg2)


def kernel(coord, feat, Wq, Wk, Wv, Wp1, bp1, Wp2, bp2, Wg1, bg1, Wg2, bg2):
    cp8 = jnp.zeros((N, 8), jnp.float32).at[:, :3].set(coord)
    cp16 = jnp.zeros((N, 16), jnp.float32).at[:, :3].set(coord)
    cpT = cp8.T
    Wp1p = jnp.zeros((8, H), jnp.float32).at[:3, :].set(Wp1)
    bp1r, bp2r = bp1.reshape(1, H), bp2.reshape(1, H)
    bg1r, bg2r = bg1.reshape(1, H), bg2.reshape(1, H)

    table, q = pl.pallas_call(
        _table_body,
        grid=(NBLK,),
        in_specs=[
            pl.BlockSpec((BQ, D), lambda i: (i, 0)),
            pl.BlockSpec((BQ, 16), lambda i: (i, 0)),
            pl.BlockSpec((D, H), lambda i: (0, 0)),
            pl.BlockSpec((D, H), lambda i: (0, 0)),
            pl.BlockSpec((D, H), lambda i: (0, 0)),
        ],
        out_specs=[
            pl.BlockSpec((BQ, TW), lambda i: (i, 0)),
            pl.BlockSpec((BQ, H), lambda i: (i, 0)),
        ],
        out_shape=[
            jax.ShapeDtypeStruct((N, TW), jnp.float32),
            jax.ShapeDtypeStruct((N, H), jnp.float32),
        ],
    )(feat, cp16, Wq, Wk, Wv)

    gath = []
    for h in range(_NSPLIT):
        idx = _topk_half(cp8, cpT, h)
        gath.append(_sc_gather(table, idx.reshape(_HROWS)))
    outs = [_attn_half(g, q, cp8, feat, Wp1p, bp1r, bp2r, Wp2,
                       Wg1, bg1r, Wg2, bg2r, h)
            for h, g in enumerate(gath)]
    return jnp.concatenate(outs, axis=0)


# speedup vs baseline: 1.0764x; 1.0764x over previous
import jax
import jax.numpy as jnp
from jax import lax
from jax.experimental import pallas as pl
from jax.experimental.pallas import tpu as pltpu
from jax.experimental.pallas import tpu_sc as plsc

N = 4096
D = 256
H = 256
KNN = 16
BQ = 256
TW = 2 * H + 128
NBLK = N // BQ

_SC_CORES = 2
_SC_SUBCORES = 16
_NW = _SC_CORES * _SC_SUBCORES
_BPW = (N * KNN) // _NW
_CH = 64


_IMAX = 0x7FFFFFFF
_NG = 32
_NL = 128
_SDEPTH = 4


def _table_body(feat_b, cp16_b, Wq, Wk, Wv, table_b, q_b):
    f = feat_b[...]
    q_b[...] = jnp.dot(f, Wq[...], preferred_element_type=jnp.float32)
    table_b[:, 0:H] = jnp.dot(f, Wk[...], preferred_element_type=jnp.float32)
    table_b[:, H:2 * H] = jnp.dot(f, Wv[...], preferred_element_type=jnp.float32)
    table_b[:, 2 * H:2 * H + 16] = cp16_b[...]


def _topk_body(cp_b, cpT, idx_b):
    c = cp_b[...]
    ct = cpT[...]
    g = jnp.dot(c, ct, preferred_element_type=jnp.float32)
    sqb = jnp.sum(c * c, axis=1, keepdims=True)
    sqr = jnp.sum(ct * ct, axis=0, keepdims=True)
    d = sqb + sqr - 2.0 * g
    bits = lax.bitcast_convert_type(jnp.maximum(d, 0.0), jnp.int32)
    d3 = bits.reshape(BQ, _NG, _NL)
    giota = lax.broadcasted_iota(jnp.int32, (BQ, _NG, _NL), 1)
    cur = d3
    svals, sgidx = [], []
    for _ in range(_SDEPTH):
        m = jnp.min(cur, axis=1)
        ag = jnp.min(jnp.where(cur == m[:, None, :], giota, _NG), axis=1)
        svals.append(m)
        sgidx.append(ag)
        cur = jnp.where(giota == ag[:, None, :], _IMAX, cur)
    m5 = jnp.min(cur, axis=1)
    s0, s1, s2, s3 = svals
    g0, g1, g2, g3 = sgidx
    liota = lax.broadcasted_iota(jnp.int32, (BQ, _NL), 1)
    cnt = jnp.zeros((BQ, _NL), jnp.int32)
    cols = []
    mn = None
    for _ in range(KNN):
        mn = jnp.min(s0, axis=1, keepdims=True)
        gi = jnp.min(jnp.where(s0 == mn, g0 * _NL + liota, N),
                     axis=1, keepdims=True)
        win = liota == (gi & (_NL - 1))
        cols.append(gi)
        s0 = jnp.where(win, s1, s0)
        g0 = jnp.where(win, g1, g0)
        s1 = jnp.where(win, s2, s1)
        g1 = jnp.where(win, g2, g1)
        s2 = jnp.where(win, s3, s2)
        g2 = jnp.where(win, g3, g2)
        s3 = jnp.where(win, _IMAX, s3)
        cnt = cnt + win.astype(jnp.int32)
    idx_b[...] = jnp.concatenate(cols, axis=1)
    bad = jnp.any((cnt >= _SDEPTH) & (m5 <= mn))

    @pl.when(bad)
    def _fallback():
        dd = bits
        iota = lax.broadcasted_iota(jnp.int32, (BQ, N), 1)
        fcols = []
        for _ in range(KNN):
            fmn = jnp.min(dd, axis=1, keepdims=True)
            am = jnp.min(jnp.where(dd == fmn, iota, N), axis=1, keepdims=True)
            fcols.append(am)
            dd = jnp.where(iota == am, _IMAX, dd)
        idx_b[...] = jnp.concatenate(fcols, axis=1)


def _stage_c_body(G_b, q_b, cp_b, feat_b, Wp1p, bp1, Wp2, bp2,
                  Wg1, bg1, Wg2, bg2, out_b):
    Gv = G_b[...]
    kj = Gv[:, 0:H]
    vj = Gv[:, H:2 * H]
    cj = Gv[:, 2 * H:2 * H + 8]
    cq = cp_b[...]
    rel = (cq[:, None, :] - cj.reshape(BQ, KNN, 8)).reshape(BQ * KNN, 8)
    t = jnp.maximum(
        jnp.dot(rel, Wp1p[...], preferred_element_type=jnp.float32) + bp1[...],
        0.0)
    delta = jnp.dot(t, Wp2[...], preferred_element_type=jnp.float32) + bp2[...]
    qv = q_b[...]
    gg = (qv[:, None, :] - kj.reshape(BQ, KNN, H)
          + delta.reshape(BQ, KNN, H)).reshape(BQ * KNN, H)
    h = jnp.maximum(
        jnp.dot(gg, Wg1[...], preferred_element_type=jnp.float32) + bg1[...],
        0.0)
    logits = (jnp.dot(h, Wg2[...], preferred_element_type=jnp.float32)
              + bg2[...]).reshape(BQ, KNN, H)
    m = jnp.max(logits, axis=1, keepdims=True)
    e = jnp.exp(logits - m)
    s = jnp.sum(e, axis=1, keepdims=True)
    attn = e / s
    wv = vj.reshape(BQ, KNN, H) + delta.reshape(BQ, KNN, H)
    out_b[...] = feat_b[...] + jnp.sum(attn * wv, axis=1)


_NSPLIT = 2
_HB = NBLK // _NSPLIT
_HROWS = (N // _NSPLIT) * KNN
_BPW = _HROWS // _NW


def _sc_gather_body(table_hbm, idx_hbm, out_hbm, idx_v, buf, sem):
    wid = lax.axis_index("s") * _SC_CORES + lax.axis_index("c")
    base = wid * _BPW
    pltpu.sync_copy(idx_hbm.at[pl.ds(base, _BPW)], idx_v)

    def chunk(c, carry):
        off = c * _CH
        pltpu.async_copy(table_hbm.at[idx_v.at[pl.ds(off, _CH)]],
                         buf, sem).wait()
        pltpu.sync_copy(buf, out_hbm.at[pl.ds(base + off, _CH)])
        return carry

    lax.fori_loop(0, _BPW // _CH, chunk, 0)


def _sc_gather(table, idx_flat):
    mesh = plsc.VectorSubcoreMesh(core_axis_name="c", subcore_axis_name="s")
    f = pl.kernel(
        _sc_gather_body,
        mesh=mesh,
        out_type=jax.ShapeDtypeStruct((_HROWS, TW), jnp.float32),
        scratch_types=[
            pltpu.VMEM((_BPW,), jnp.int32),
            pltpu.VMEM((_CH, TW), jnp.float32),
            pltpu.SemaphoreType.DMA,
        ],
    )
    return f(table, idx_flat)


def _topk_half(cp8, cpT, half):
    return pl.pallas_call(
        _topk_body,
        grid=(_HB,),
        in_specs=[
            pl.BlockSpec((BQ, 8), lambda i: (i + half * _HB, 0)),
            pl.BlockSpec((8, N), lambda i: (0, 0)),
        ],
        out_specs=pl.BlockSpec((BQ, KNN), lambda i: (i, 0)),
        out_shape=jax.ShapeDtypeStruct((N // _NSPLIT, KNN), jnp.int32),
    )(cp8, cpT)


def _attn_half(gathered, q, cp8, feat, Wp1p, bp1, bp2, Wp2, Wg1, bg1,
               Wg2, bg2, half):
    return pl.pallas_call(
        _stage_c_body,
        grid=(_HB,),
        in_specs=[
            pl.BlockSpec((BQ * KNN, TW), lambda i: (i, 0)),
            pl.BlockSpec((BQ, H), lambda i: (i + half * _HB, 0)),
            pl.BlockSpec((BQ, 8), lambda i: (i + half * _HB, 0)),
            pl.BlockSpec((BQ, D), lambda i: (i + half * _HB, 0)),
            pl.BlockSpec((8, H), lambda i: (0, 0)),
            pl.BlockSpec((1, H), lambda i: (0, 0)),
            pl.BlockSpec((H, H), lambda i: (0, 0)),
            pl.BlockSpec((1, H), lambda i: (0, 0)),
            pl.BlockSpec((H, H), lambda i: (0, 0)),
            pl.BlockSpec((1, H), lambda i: (0, 0)),
            pl.BlockSpec((H, H), lambda i: (0, 0)),
            pl.BlockSpec((1, H), lambda i: (0, 0)),
        ],
        out_specs=pl.BlockSpec((BQ, H), lambda i: (i, 0)),
        out_shape=jax.ShapeDtypeStruct((N // _NSPLIT, H), jnp.float32),
    )(gathered, q, cp8, feat,
      Wp1p, bp1, Wp2, bp2, Wg1, bg1, Wg2, bg2)


def kernel(coord, feat, Wq, Wk, Wv, Wp1, bp1, Wp2, bp2, Wg1, bg1, Wg2, bg2):
    cp8 = jnp.zeros((N, 8), jnp.float32).at[:, :3].set(coord)
    cp16 = jnp.zeros((N, 16), jnp.float32).at[:, :3].set(coord)
    cpT = cp8.T
    Wp1p = jnp.zeros((8, H), jnp.float32).at[:3, :].set(Wp1)
    bp1r, bp2r = bp1.reshape(1, H), bp2.reshape(1, H)
    bg1r, bg2r = bg1.reshape(1, H), bg2.reshape(1, H)

    table, q = pl.pallas_call(
        _table_body,
        grid=(NBLK,),
        in_specs=[
            pl.BlockSpec((BQ, D), lambda i: (i, 0)),
            pl.BlockSpec((BQ, 16), lambda i: (i, 0)),
            pl.BlockSpec((D, H), lambda i: (0, 0)),
            pl.BlockSpec((D, H), lambda i: (0, 0)),
            pl.BlockSpec((D, H), lambda i: (0, 0)),
        ],
        out_specs=[
            pl.BlockSpec((BQ, TW), lambda i: (i, 0)),
            pl.BlockSpec((BQ, H), lambda i: (i, 0)),
        ],
        out_shape=[
            jax.ShapeDtypeStruct((N, TW), jnp.float32),
            jax.ShapeDtypeStruct((N, H), jnp.float32),
        ],
    )(feat, cp16, Wq, Wk, Wv)

    gath = []
    for h in range(_NSPLIT):
        idx = _topk_half(cp8, cpT, h)
        gath.append(_sc_gather(table, idx.reshape(_HROWS)))
    outs = [_attn_half(g, q, cp8, feat, Wp1p, bp1r, bp2r, Wp2,
                       Wg1, bg1r, Wg2, bg2r, h)
            for h, g in enumerate(gath)]
    return jnp.concatenate(outs, axis=0)


# speedup vs baseline: 1.2967x; 1.2047x over previous
import jax
import jax.numpy as jnp
from jax import lax
from jax.experimental import pallas as pl
from jax.experimental.pallas import tpu as pltpu
from jax.experimental.pallas import tpu_sc as plsc

N = 4096
D = 256
H = 256
KNN = 16
BQ = 256
TW = 2 * H + 128
NBLK = N // BQ

_SC_CORES = 2
_SC_SUBCORES = 16
_NW = _SC_CORES * _SC_SUBCORES
_BPW = (N * KNN) // _NW
_CH = 64


_IMAX = 0x7FFFFFFF
_NG = 32
_NL = 128
_SDEPTH = 4


def _table_body(feat_b, cp16_b, Wq, Wk, Wv, table_b, q_b):
    f = feat_b[...]
    q_b[...] = jnp.dot(f, Wq[...], preferred_element_type=jnp.float32)
    table_b[:, 0:H] = jnp.dot(f, Wk[...], preferred_element_type=jnp.float32)
    table_b[:, H:2 * H] = jnp.dot(f, Wv[...], preferred_element_type=jnp.float32)
    table_b[:, 2 * H:2 * H + 16] = cp16_b[...]


def _topk_body(cp_b, cpT, idx_b):
    c = cp_b[...]
    ct = cpT[...]
    g = jnp.dot(c, ct, preferred_element_type=jnp.float32)
    sqb = jnp.sum(c * c, axis=1, keepdims=True)
    sqr = jnp.sum(ct * ct, axis=0, keepdims=True)
    d = sqb + sqr - 2.0 * g
    bits = lax.bitcast_convert_type(jnp.maximum(d, 0.0), jnp.int32)
    sv = [jnp.full((BQ, _NL), _IMAX, jnp.int32) for _ in range(_SDEPTH)]
    sg = [jnp.zeros((BQ, _NL), jnp.int32) for _ in range(_SDEPTH)]
    m5 = jnp.full((BQ, _NL), _IMAX, jnp.int32)
    for grp in range(_NG):
        xv = bits[:, grp * _NL:(grp + 1) * _NL]
        xg = jnp.full((BQ, _NL), grp, jnp.int32)
        for lvl in range(_SDEPTH):
            cc = xv < sv[lvl]
            sv[lvl], xv = (jnp.where(cc, xv, sv[lvl]),
                           jnp.where(cc, sv[lvl], xv))
            sg[lvl], xg = (jnp.where(cc, xg, sg[lvl]),
                           jnp.where(cc, sg[lvl], xg))
        m5 = jnp.minimum(m5, xv)
    s0, s1, s2, s3 = sv
    g0, g1, g2, g3 = sg
    liota = lax.broadcasted_iota(jnp.int32, (BQ, _NL), 1)
    cnt = jnp.zeros((BQ, _NL), jnp.int32)
    cols = []
    mn = None
    for _ in range(KNN):
        mn = jnp.min(s0, axis=1, keepdims=True)
        gi = jnp.min(jnp.where(s0 == mn, g0 * _NL + liota, N),
                     axis=1, keepdims=True)
        win = liota == (gi & (_NL - 1))
        cols.append(gi)
        s0 = jnp.where(win, s1, s0)
        g0 = jnp.where(win, g1, g0)
        s1 = jnp.where(win, s2, s1)
        g1 = jnp.where(win, g2, g1)
        s2 = jnp.where(win, s3, s2)
        g2 = jnp.where(win, g3, g2)
        s3 = jnp.where(win, _IMAX, s3)
        cnt = cnt + win.astype(jnp.int32)
    idx_b[...] = jnp.concatenate(cols, axis=1)
    bad = jnp.any((cnt >= _SDEPTH) & (m5 <= mn))

    @pl.when(bad)
    def _fallback():
        dd = bits
        iota = lax.broadcasted_iota(jnp.int32, (BQ, N), 1)
        fcols = []
        for _ in range(KNN):
            fmn = jnp.min(dd, axis=1, keepdims=True)
            am = jnp.min(jnp.where(dd == fmn, iota, N), axis=1, keepdims=True)
            fcols.append(am)
            dd = jnp.where(iota == am, _IMAX, dd)
        idx_b[...] = jnp.concatenate(fcols, axis=1)


def _stage_c_body(G_b, q_b, cp_b, feat_b, Wp1p, bp1, Wp2, bp2,
                  Wg1, bg1, Wg2, bg2, out_b):
    Gv = G_b[...]
    kj = Gv[:, 0:H]
    vj = Gv[:, H:2 * H]
    cj = Gv[:, 2 * H:2 * H + 8]
    cq = cp_b[...]
    rel = (cq[:, None, :] - cj.reshape(BQ, KNN, 8)).reshape(BQ * KNN, 8)
    t = jnp.maximum(
        jnp.dot(rel, Wp1p[...], preferred_element_type=jnp.float32) + bp1[...],
        0.0)
    delta = jnp.dot(t, Wp2[...], preferred_element_type=jnp.float32) + bp2[...]
    qv = q_b[...]
    gg = (qv[:, None, :] - kj.reshape(BQ, KNN, H)
          + delta.reshape(BQ, KNN, H)).reshape(BQ * KNN, H)
    h = jnp.maximum(
        jnp.dot(gg, Wg1[...], preferred_element_type=jnp.float32) + bg1[...],
        0.0)
    logits = (jnp.dot(h, Wg2[...], preferred_element_type=jnp.float32)
              + bg2[...]).reshape(BQ, KNN, H)
    m = jnp.max(logits, axis=1, keepdims=True)
    e = jnp.exp(logits - m)
    s = jnp.sum(e, axis=1, keepdims=True)
    attn = e / s
    wv = vj.reshape(BQ, KNN, H) + delta.reshape(BQ, KNN, H)
    out_b[...] = feat_b[...] + jnp.sum(attn * wv, axis=1)


_NSPLIT = 2
_HB = NBLK // _NSPLIT
_HROWS = (N // _NSPLIT) * KNN
_BPW = _HROWS // _NW


def _sc_gather_body(table_hbm, idx_hbm, out_hbm, idx_v, buf, sem):
    wid = lax.axis_index("s") * _SC_CORES + lax.axis_index("c")
    base = wid * _BPW
    pltpu.sync_copy(idx_hbm.at[pl.ds(base, _BPW)], idx_v)

    def chunk(c, carry):
        off = c * _CH
        pltpu.async_copy(table_hbm.at[idx_v.at[pl.ds(off, _CH)]],
                         buf, sem).wait()
        pltpu.sync_copy(buf, out_hbm.at[pl.ds(base + off, _CH)])
        return carry

    lax.fori_loop(0, _BPW // _CH, chunk, 0)


def _sc_gather(table, idx_flat):
    mesh = plsc.VectorSubcoreMesh(core_axis_name="c", subcore_axis_name="s")
    f = pl.kernel(
        _sc_gather_body,
        mesh=mesh,
        out_type=jax.ShapeDtypeStruct((_HROWS, TW), jnp.float32),
        scratch_types=[
            pltpu.VMEM((_BPW,), jnp.int32),
            pltpu.VMEM((_CH, TW), jnp.float32),
            pltpu.SemaphoreType.DMA,
        ],
    )
    return f(table, idx_flat)


def _topk_half(cp8, cpT, half):
    return pl.pallas_call(
        _topk_body,
        grid=(_HB,),
        in_specs=[
            pl.BlockSpec((BQ, 8), lambda i: (i + half * _HB, 0)),
            pl.BlockSpec((8, N), lambda i: (0, 0)),
        ],
        out_specs=pl.BlockSpec((BQ, KNN), lambda i: (i, 0)),
        out_shape=jax.ShapeDtypeStruct((N // _NSPLIT, KNN), jnp.int32),
    )(cp8, cpT)


def _attn_half(gathered, q, cp8, feat, Wp1p, bp1, bp2, Wp2, Wg1, bg1,
               Wg2, bg2, half):
    return pl.pallas_call(
        _stage_c_body,
        grid=(_HB,),
        in_specs=[
            pl.BlockSpec((BQ * KNN, TW), lambda i: (i, 0)),
            pl.BlockSpec((BQ, H), lambda i: (i + half * _HB, 0)),
            pl.BlockSpec((BQ, 8), lambda i: (i + half * _HB, 0)),
            pl.BlockSpec((BQ, D), lambda i: (i + half * _HB, 0)),
            pl.BlockSpec((8, H), lambda i: (0, 0)),
            pl.BlockSpec((1, H), lambda i: (0, 0)),
            pl.BlockSpec((H, H), lambda i: (0, 0)),
            pl.BlockSpec((1, H), lambda i: (0, 0)),
            pl.BlockSpec((H, H), lambda i: (0, 0)),
            pl.BlockSpec((1, H), lambda i: (0, 0)),
            pl.BlockSpec((H, H), lambda i: (0, 0)),
            pl.BlockSpec((1, H), lambda i: (0, 0)),
        ],
        out_specs=pl.BlockSpec((BQ, H), lambda i: (i, 0)),
        out_shape=jax.ShapeDtypeStruct((N // _NSPLIT, H), jnp.float32),
    )(gathered, q, cp8, feat,
      Wp1p, bp1, Wp2, bp2, Wg1, bg1, Wg2, bg2)


def kernel(coord, feat, Wq, Wk, Wv, Wp1, bp1, Wp2, bp2, Wg1, bg1, Wg2, bg2):
    cp8 = jnp.zeros((N, 8), jnp.float32).at[:, :3].set(coord)
    cp16 = jnp.zeros((N, 16), jnp.float32).at[:, :3].set(coord)
    cpT = cp8.T
    Wp1p = jnp.zeros((8, H), jnp.float32).at[:3, :].set(Wp1)
    bp1r, bp2r = bp1.reshape(1, H), bp2.reshape(1, H)
    bg1r, bg2r = bg1.reshape(1, H), bg2.reshape(1, H)

    table, q = pl.pallas_call(
        _table_body,
        grid=(NBLK,),
        in_specs=[
            pl.BlockSpec((BQ, D), lambda i: (i, 0)),
            pl.BlockSpec((BQ, 16), lambda i: (i, 0)),
            pl.BlockSpec((D, H), lambda i: (0, 0)),
            pl.BlockSpec((D, H), lambda i: (0, 0)),
            pl.BlockSpec((D, H), lambda i: (0, 0)),
        ],
        out_specs=[
            pl.BlockSpec((BQ, TW), lambda i: (i, 0)),
            pl.BlockSpec((BQ, H), lambda i: (i, 0)),
        ],
        out_shape=[
            jax.ShapeDtypeStruct((N, TW), jnp.float32),
            jax.ShapeDtypeStruct((N, H), jnp.float32),
        ],
    )(feat, cp16, Wq, Wk, Wv)

    gath = []
    for h in range(_NSPLIT):
        idx = _topk_half(cp8, cpT, h)
        gath.append(_sc_gather(table, idx.reshape(_HROWS)))
    outs = [_attn_half(g, q, cp8, feat, Wp1p, bp1r, bp2r, Wp2,
                       Wg1, bg1r, Wg2, bg2r, h)
            for h, g in enumerate(gath)]
    return jnp.concatenate(outs, axis=0)


# speedup vs baseline: 1.3002x; 1.0027x over previous
import jax
import jax.numpy as jnp
from jax import lax
from jax.experimental import pallas as pl
from jax.experimental.pallas import tpu as pltpu
from jax.experimental.pallas import tpu_sc as plsc

N = 4096
D = 256
H = 256
KNN = 16
BQ = 256
TW = 2 * H + 128
NBLK = N // BQ

_SC_CORES = 2
_SC_SUBCORES = 16
_NW = _SC_CORES * _SC_SUBCORES
_BPW = (N * KNN) // _NW
_CH = 64


_IMAX = 0x7FFFFFFF
_NG = 32
_NL = 128
_SDEPTH = 4


def _table_body(feat_b, cp16_b, Wq, Wk, Wv, table_b, q_b):
    f = feat_b[...]
    q_b[...] = jnp.dot(f, Wq[...], preferred_element_type=jnp.float32)
    table_b[:, 0:H] = jnp.dot(f, Wk[...], preferred_element_type=jnp.float32)
    table_b[:, H:2 * H] = jnp.dot(f, Wv[...], preferred_element_type=jnp.float32)
    table_b[:, 2 * H:2 * H + 16] = cp16_b[...]


def _topk_body(cp_b, cpT, idx_b):
    c = cp_b[...]
    ct = cpT[...]
    g = jnp.dot(c, ct, preferred_element_type=jnp.float32)
    sqb = jnp.sum(c * c, axis=1, keepdims=True)
    sqr = jnp.sum(ct * ct, axis=0, keepdims=True)
    d = sqb + sqr - 2.0 * g
    bits = lax.bitcast_convert_type(jnp.maximum(d, 0.0), jnp.int32)
    liota = lax.broadcasted_iota(jnp.int32, (BQ, _NL), 1)
    sv = [jnp.full((BQ, _NL), _IMAX, jnp.int32) for _ in range(_SDEPTH)]
    sg = [jnp.zeros((BQ, _NL), jnp.int32) for _ in range(_SDEPTH)]
    m5 = jnp.full((BQ, _NL), _IMAX, jnp.int32)
    for grp in range(_NG):
        xv = bits[:, grp * _NL:(grp + 1) * _NL]
        xg = liota + grp * _NL
        for lvl in range(_SDEPTH):
            cc = xv < sv[lvl]
            sg[lvl], xg_new = (jnp.where(cc, xg, sg[lvl]),
                               jnp.where(cc, sg[lvl], xg))
            sv[lvl], xv = (jnp.where(cc, xv, sv[lvl]),
                           jnp.where(cc, sv[lvl], xv))
            if lvl < _SDEPTH - 1:
                xg = xg_new
        m5 = jnp.minimum(m5, xv)
    s0, s1, s2, s3 = sv
    g0, g1, g2, g3 = sg
    cols = []
    mn = None
    for _ in range(KNN):
        mn = jnp.min(s0, axis=1, keepdims=True)
        gi = jnp.min(jnp.where(s0 == mn, g0, N), axis=1, keepdims=True)
        win = liota == (gi & (_NL - 1))
        cols.append(gi)
        s0 = jnp.where(win, s1, s0)
        g0 = jnp.where(win, g1, g0)
        s1 = jnp.where(win, s2, s1)
        g1 = jnp.where(win, g2, g1)
        s2 = jnp.where(win, s3, s2)
        g2 = jnp.where(win, g3, g2)
        s3 = jnp.where(win, _IMAX, s3)
    idx_b[...] = jnp.concatenate(cols, axis=1)
    bad = jnp.any((s0 == _IMAX) & (m5 <= mn))

    @pl.when(bad)
    def _fallback():
        dd = bits
        iota = lax.broadcasted_iota(jnp.int32, (BQ, N), 1)
        fcols = []
        for _ in range(KNN):
            fmn = jnp.min(dd, axis=1, keepdims=True)
            am = jnp.min(jnp.where(dd == fmn, iota, N), axis=1, keepdims=True)
            fcols.append(am)
            dd = jnp.where(iota == am, _IMAX, dd)
        idx_b[...] = jnp.concatenate(fcols, axis=1)


def _stage_c_body(G_b, q_b, cp_b, feat_b, Wp1p, bp1, Wp2, bp2,
                  Wg1, bg1, Wg2, bg2, out_b):
    Gv = G_b[...]
    kj = Gv[:, 0:H]
    vj = Gv[:, H:2 * H]
    cj = Gv[:, 2 * H:2 * H + 8]
    cq = cp_b[...]
    rel = (cq[:, None, :] - cj.reshape(BQ, KNN, 8)).reshape(BQ * KNN, 8)
    t = jnp.maximum(
        jnp.dot(rel, Wp1p[...], preferred_element_type=jnp.float32) + bp1[...],
        0.0)
    delta = jnp.dot(t, Wp2[...], preferred_element_type=jnp.float32) + bp2[...]
    qv = q_b[...]
    gg = (qv[:, None, :] - kj.reshape(BQ, KNN, H)
          + delta.reshape(BQ, KNN, H)).reshape(BQ * KNN, H)
    h = jnp.maximum(
        jnp.dot(gg, Wg1[...], preferred_element_type=jnp.float32) + bg1[...],
        0.0)
    logits = (jnp.dot(h, Wg2[...], preferred_element_type=jnp.float32)
              + bg2[...]).reshape(BQ, KNN, H)
    m = jnp.max(logits, axis=1, keepdims=True)
    e = jnp.exp(logits - m)
    s = jnp.sum(e, axis=1, keepdims=True)
    attn = e / s
    wv = vj.reshape(BQ, KNN, H) + delta.reshape(BQ, KNN, H)
    out_b[...] = feat_b[...] + jnp.sum(attn * wv, axis=1)


_NSPLIT = 2
_HB = NBLK // _NSPLIT
_HROWS = (N // _NSPLIT) * KNN
_BPW = _HROWS // _NW


def _sc_gather_body(table_hbm, idx_hbm, out_hbm, idx_v, buf, sem):
    wid = lax.axis_index("s") * _SC_CORES + lax.axis_index("c")
    base = wid * _BPW
    pltpu.sync_copy(idx_hbm.at[pl.ds(base, _BPW)], idx_v)

    def chunk(c, carry):
        off = c * _CH
        pltpu.async_copy(table_hbm.at[idx_v.at[pl.ds(off, _CH)]],
                         buf, sem).wait()
        pltpu.sync_copy(buf, out_hbm.at[pl.ds(base + off, _CH)])
        return carry

    lax.fori_loop(0, _BPW // _CH, chunk, 0)


def _sc_gather(table, idx_flat):
    mesh = plsc.VectorSubcoreMesh(core_axis_name="c", subcore_axis_name="s")
    f = pl.kernel(
        _sc_gather_body,
        mesh=mesh,
        out_type=jax.ShapeDtypeStruct((_HROWS, TW), jnp.float32),
        scratch_types=[
            pltpu.VMEM((_BPW,), jnp.int32),
            pltpu.VMEM((_CH, TW), jnp.float32),
            pltpu.SemaphoreType.DMA,
        ],
    )
    return f(table, idx_flat)


def _topk_half(cp8, cpT, half):
    return pl.pallas_call(
        _topk_body,
        grid=(_HB,),
        in_specs=[
            pl.BlockSpec((BQ, 8), lambda i: (i + half * _HB, 0)),
            pl.BlockSpec((8, N), lambda i: (0, 0)),
        ],
        out_specs=pl.BlockSpec((BQ, KNN), lambda i: (i, 0)),
        out_shape=jax.ShapeDtypeStruct((N // _NSPLIT, KNN), jnp.int32),
    )(cp8, cpT)


def _attn_half(gathered, q, cp8, feat, Wp1p, bp1, bp2, Wp2, Wg1, bg1,
               Wg2, bg2, half):
    return pl.pallas_call(
        _stage_c_body,
        grid=(_HB,),
        in_specs=[
            pl.BlockSpec((BQ * KNN, TW), lambda i: (i, 0)),
            pl.BlockSpec((BQ, H), lambda i: (i + half * _HB, 0)),
            pl.BlockSpec((BQ, 8), lambda i: (i + half * _HB, 0)),
            pl.BlockSpec((BQ, D), lambda i: (i + half * _HB, 0)),
            pl.BlockSpec((8, H), lambda i: (0, 0)),
            pl.BlockSpec((1, H), lambda i: (0, 0)),
            pl.BlockSpec((H, H), lambda i: (0, 0)),
            pl.BlockSpec((1, H), lambda i: (0, 0)),
            pl.BlockSpec((H, H), lambda i: (0, 0)),
            pl.BlockSpec((1, H), lambda i: (0, 0)),
            pl.BlockSpec((H, H), lambda i: (0, 0)),
            pl.BlockSpec((1, H), lambda i: (0, 0)),
        ],
        out_specs=pl.BlockSpec((BQ, H), lambda i: (i, 0)),
        out_shape=jax.ShapeDtypeStruct((N // _NSPLIT, H), jnp.float32),
    )(gathered, q, cp8, feat,
      Wp1p, bp1, Wp2, bp2, Wg1, bg1, Wg2, bg2)


def kernel(coord, feat, Wq, Wk, Wv, Wp1, bp1, Wp2, bp2, Wg1, bg1, Wg2, bg2):
    cp8 = jnp.zeros((N, 8), jnp.float32).at[:, :3].set(coord)
    cp16 = jnp.zeros((N, 16), jnp.float32).at[:, :3].set(coord)
    cpT = cp8.T
    Wp1p = jnp.zeros((8, H), jnp.float32).at[:3, :].set(Wp1)
    bp1r, bp2r = bp1.reshape(1, H), bp2.reshape(1, H)
    bg1r, bg2r = bg1.reshape(1, H), bg2.reshape(1, H)

    table, q = pl.pallas_call(
        _table_body,
        grid=(NBLK,),
        in_specs=[
            pl.BlockSpec((BQ, D), lambda i: (i, 0)),
            pl.BlockSpec((BQ, 16), lambda i: (i, 0)),
            pl.BlockSpec((D, H), lambda i: (0, 0)),
            pl.BlockSpec((D, H), lambda i: (0, 0)),
            pl.BlockSpec((D, H), lambda i: (0, 0)),
        ],
        out_specs=[
            pl.BlockSpec((BQ, TW), lambda i: (i, 0)),
            pl.BlockSpec((BQ, H), lambda i: (i, 0)),
        ],
        out_shape=[
            jax.ShapeDtypeStruct((N, TW), jnp.float32),
            jax.ShapeDtypeStruct((N, H), jnp.float32),
        ],
    )(feat, cp16, Wq, Wk, Wv)

    gath = []
    for h in range(_NSPLIT):
        idx = _topk_half(cp8, cpT, h)
        gath.append(_sc_gather(table, idx.reshape(_HROWS)))
    outs = [_attn_half(g, q, cp8, feat, Wp1p, bp1r, bp2r, Wp2,
                       Wg1, bg1r, Wg2, bg2r, h)
            for h, g in enumerate(gath)]
    return jnp.concatenate(outs, axis=0)
